# Initial kernel scaffold; baseline (speedup 1.0000x reference)
import jax
import jax.numpy as jnp
from jax.experimental import pallas as pl


def kernel(Wv, bv, Ws, bs, gv, betav, ge, betae, weight, a_mean_b, s_gather_b, v_in, e_in):
    raise NotImplementedError("write your pallas kernel here")


# speedup vs baseline: 1.0809x; 1.0809x over previous
import functools

import jax
import jax.numpy as jnp
from jax.experimental import pallas as pl
from jax.experimental.pallas import tpu as pltpu

_LEAKY_SLOPE = 0.2
_BN_EPS = 1e-5
_CB = 8

_CELL_ARCH = (
    (0, 1, 0), (0, 2, 1), (1, 2, 2), (1, 3, 3), (2, 3, 4), (0, 4, 5),
    (3, 4, 6),
)
_NB_NODES = 4


def _build_links():
    d = {}
    for src, dst, w in _CELL_ARCH:
        d.setdefault(dst, []).append((src, w))
    return tuple(tuple(d[k]) for k in range(1, _NB_NODES + 1))


_LINKS = _build_links()

_ROW_BV, _ROW_GV, _ROW_BETAV, _ROW_BS, _ROW_GE, _ROW_BETAE = range(6)


def _kernel_body(wt_ref, a_ref, g_ref, v_ref, e_ref, wv_ref, ws_ref, pp_ref,
                 vout_ref, eout_ref, *, node_dim, edge_dim, slope, eps):
    d, de = node_dim, edge_dim
    m = e_ref.shape[1]

    bv = pp_ref[_ROW_BV:_ROW_BV + 1, :d]
    gv = pp_ref[_ROW_GV:_ROW_GV + 1, :d]
    betav = pp_ref[_ROW_BETAV:_ROW_BETAV + 1, :d]
    bs = pp_ref[_ROW_BS:_ROW_BS + 1, :de]
    ge = pp_ref[_ROW_GE:_ROW_GE + 1, :de]
    betae = pp_ref[_ROW_BETAE:_ROW_BETAE + 1, :de]

    wv = wv_ref[...]
    ws = ws_ref[...]

    def bn_leaky_res(h, g, b, res):
        inv_n = 1.0 / h.shape[0]
        mean = jnp.sum(h, axis=0, keepdims=True) * inv_n
        diff = h - mean
        var = jnp.sum(diff * diff, axis=0, keepdims=True) * inv_n
        hn = diff * jax.lax.rsqrt(var + eps) * g + b
        return jnp.where(hn >= 0, hn, slope * hn) + res

    for c in range(_CB):
        a_mean = a_ref[c]
        v_in = v_ref[c]
        e_in = e_ref[c]

        states = [v_in]
        aggs = {}

        def agg_of(s):
            if s not in aggs:
                aggs[s] = jnp.dot(a_mean, states[s],
                                  preferred_element_type=jnp.float32)
            return aggs[s]

        for dst_links in _LINKS:
            acc = None
            for s, w in dst_links:
                term = wt_ref[w, 1] * states[s] + wt_ref[w, 2] * agg_of(s)
                acc = term if acc is None else acc + term
            states.append(acc)

        conc = jnp.concatenate(states[1:], axis=1)
        v_lin = jnp.dot(conc, wv,
                        preferred_element_type=jnp.float32) + bv

        vg = jnp.dot(g_ref[c], v_lin, preferred_element_type=jnp.float32)
        e_act = jnp.where(e_in >= 0, e_in, slope * e_in)
        cat = jnp.concatenate([vg[:m], e_act, vg[m:]], axis=1)
        e_lin = jnp.dot(cat, ws, preferred_element_type=jnp.float32) + bs

        vout_ref[c] = bn_leaky_res(v_lin, gv, betav, v_in)
        eout_ref[c] = bn_leaky_res(e_lin, ge, betae, e_in)


def _pack_params(bv, gv, betav, bs, ge, betae, d, de):
    pp = jnp.zeros((8, 128), jnp.float32)
    pp = pp.at[_ROW_BV, :d].set(bv.reshape(-1))
    pp = pp.at[_ROW_GV, :d].set(gv.reshape(-1))
    pp = pp.at[_ROW_BETAV, :d].set(betav.reshape(-1))
    pp = pp.at[_ROW_BS, :de].set(bs.reshape(-1))
    pp = pp.at[_ROW_GE, :de].set(ge.reshape(-1))
    pp = pp.at[_ROW_BETAE, :de].set(betae.reshape(-1))
    return pp


def kernel(Wv, bv, Ws, bs, gv, betav, ge, betae, weight,
           a_mean_b, s_gather_b, v_in, e_in):
    b, n, d = v_in.shape
    _, m, de = e_in.shape
    pp = _pack_params(bv, gv, betav, bs, ge, betae, d, de)

    body = functools.partial(_kernel_body, node_dim=d, edge_dim=de,
                             slope=_LEAKY_SLOPE, eps=_BN_EPS)

    smem = pltpu.MemorySpace.SMEM
    in_specs = [
        pl.BlockSpec(memory_space=smem),
        pl.BlockSpec((_CB, n, n), lambda i: (i, 0, 0)),
        pl.BlockSpec((_CB, 2 * m, n), lambda i: (i, 0, 0)),
        pl.BlockSpec((_CB, n, d), lambda i: (i, 0, 0)),
        pl.BlockSpec((_CB, m, de), lambda i: (i, 0, 0)),
        pl.BlockSpec(Wv.shape, lambda i: (0, 0)),
        pl.BlockSpec(Ws.shape, lambda i: (0, 0)),
        pl.BlockSpec((8, 128), lambda i: (0, 0)),
    ]
    out_specs = (
        pl.BlockSpec((_CB, n, d), lambda i: (i, 0, 0)),
        pl.BlockSpec((_CB, m, de), lambda i: (i, 0, 0)),
    )
    out_shape = (jax.ShapeDtypeStruct((b, n, d), jnp.float32),
                 jax.ShapeDtypeStruct((b, m, de), jnp.float32))

    flops_per_cell = (2 * 4 * n * n * d + 2 * n * (4 * d) * d
                      + 2 * (2 * m) * n * d + 2 * m * (2 * d + de) * de
                      + 12 * (n * d + m * de))
    bytes_accessed = 4 * (a_mean_b.size + s_gather_b.size + v_in.size
                          + e_in.size + Wv.size + Ws.size + 8 * 128
                          + b * n * d + b * m * de)

    return pl.pallas_call(
        body,
        grid=(b // _CB,),
        in_specs=in_specs,
        out_specs=out_specs,
        out_shape=out_shape,
        compiler_params=pltpu.CompilerParams(
            dimension_semantics=("parallel",)),
        cost_estimate=pl.CostEstimate(
            flops=int(b * flops_per_cell),
            transcendentals=int(b * (d + de)),
            bytes_accessed=int(bytes_accessed)),
    )(weight, a_mean_b, s_gather_b, v_in, e_in, Wv, Ws, pp)


# speedup vs baseline: 2.3056x; 2.1331x over previous
import functools

import jax
import jax.numpy as jnp
from jax.experimental import pallas as pl
from jax.experimental.pallas import tpu as pltpu

_LEAKY_SLOPE = 0.2
_BN_EPS = 1e-5
_CB = 8
_NB_NODES = 4
_DEG = _NB_NODES + 1

_CELL_ARCH = (
    (0, 1, 0), (0, 2, 1), (1, 2, 2), (1, 3, 3), (2, 3, 4), (0, 4, 5),
    (3, 4, 6),
)


def _build_links():
    d = {}
    for src, dst, w in _CELL_ARCH:
        d.setdefault(dst, []).append((src, w))
    return tuple(tuple(d[k]) for k in range(1, _NB_NODES + 1))


_LINKS = _build_links()

_ROW_BV, _ROW_GV, _ROW_BETAV, _ROW_BS, _ROW_GE, _ROW_BETAE = range(6)


def _is_zero(x):
    return isinstance(x, float) and x == 0.0


def _state_poly_coeffs(wt_ref):
    coeffs = [[1.0, 0.0, 0.0, 0.0, 0.0]]
    for dst_links in _LINKS:
        acc = [0.0] * _DEG
        for s, w in dst_links:
            w1 = wt_ref[w, 1]
            w2 = wt_ref[w, 2]
            c = coeffs[s]
            for j in range(_DEG):
                if _is_zero(c[j]):
                    continue
                acc[j] = acc[j] + w1 * c[j]
                acc[j + 1] = acc[j + 1] + w2 * c[j]
        coeffs.append(acc)
    return coeffs


def _kernel_body(wt_ref, a_ref, g_ref, v_ref, e_ref, wv_ref, ws_ref, pp_ref,
                 vout_ref, eout_ref, *, node_dim, edge_dim, slope, eps):
    d, de = node_dim, edge_dim
    m = e_ref.shape[1]

    bv = pp_ref[_ROW_BV:_ROW_BV + 1, :d]
    gv = pp_ref[_ROW_GV:_ROW_GV + 1, :d]
    betav = pp_ref[_ROW_BETAV:_ROW_BETAV + 1, :d]
    bs = pp_ref[_ROW_BS:_ROW_BS + 1, :de]
    ge = pp_ref[_ROW_GE:_ROW_GE + 1, :de]
    betae = pp_ref[_ROW_BETAE:_ROW_BETAE + 1, :de]

    ws = ws_ref[...]

    coeffs = _state_poly_coeffs(wt_ref)
    wv_slices = [wv_ref[k * d:(k + 1) * d, :] for k in range(_NB_NODES)]
    wk = []
    for j in range(_DEG):
        acc = None
        for k in range(_NB_NODES):
            c = coeffs[k + 1][j]
            if _is_zero(c):
                continue
            term = c * wv_slices[k]
            acc = term if acc is None else acc + term
        wk.append(acc)
    wk_stack = jnp.concatenate(wk, axis=0)

    def bn_leaky_res(h, g, b, res):
        inv_n = 1.0 / h.shape[0]
        mean = jnp.sum(h, axis=0, keepdims=True) * inv_n
        diff = h - mean
        var = jnp.sum(diff * diff, axis=0, keepdims=True) * inv_n
        hn = diff * jax.lax.rsqrt(var + eps) * g + b
        return jnp.where(hn >= 0, hn, slope * hn) + res

    kry = [[v_ref[c]] for c in range(_CB)]
    for _ in range(_NB_NODES):
        for c in range(_CB):
            kry[c].append(jnp.dot(a_ref[c], kry[c][-1],
                                  preferred_element_type=jnp.float32))

    v_lin = []
    for c in range(_CB):
        ck = jnp.concatenate(kry[c], axis=1)
        v_lin.append(jnp.dot(ck, wk_stack,
                             preferred_element_type=jnp.float32) + bv)

    vg = [jnp.dot(g_ref[c], v_lin[c], preferred_element_type=jnp.float32)
          for c in range(_CB)]

    e_lin = []
    for c in range(_CB):
        e_in = e_ref[c]
        e_act = jnp.where(e_in >= 0, e_in, slope * e_in)
        cat = jnp.concatenate([vg[c][:m], e_act, vg[c][m:]], axis=1)
        e_lin.append(jnp.dot(cat, ws, preferred_element_type=jnp.float32)
                     + bs)

    for c in range(_CB):
        vout_ref[c] = bn_leaky_res(v_lin[c], gv, betav, v_ref[c])
        eout_ref[c] = bn_leaky_res(e_lin[c], ge, betae, e_ref[c])


def _pack_params(bv, gv, betav, bs, ge, betae, d, de):
    pp = jnp.zeros((8, 128), jnp.float32)
    pp = pp.at[_ROW_BV, :d].set(bv.reshape(-1))
    pp = pp.at[_ROW_GV, :d].set(gv.reshape(-1))
    pp = pp.at[_ROW_BETAV, :d].set(betav.reshape(-1))
    pp = pp.at[_ROW_BS, :de].set(bs.reshape(-1))
    pp = pp.at[_ROW_GE, :de].set(ge.reshape(-1))
    pp = pp.at[_ROW_BETAE, :de].set(betae.reshape(-1))
    return pp


def kernel(Wv, bv, Ws, bs, gv, betav, ge, betae, weight,
           a_mean_b, s_gather_b, v_in, e_in):
    b, n, d = v_in.shape
    _, m, de = e_in.shape
    pp = _pack_params(bv, gv, betav, bs, ge, betae, d, de)

    body = functools.partial(_kernel_body, node_dim=d, edge_dim=de,
                             slope=_LEAKY_SLOPE, eps=_BN_EPS)

    smem = pltpu.MemorySpace.SMEM
    in_specs = [
        pl.BlockSpec(memory_space=smem),
        pl.BlockSpec((_CB, n, n), lambda i: (i, 0, 0)),
        pl.BlockSpec((_CB, 2 * m, n), lambda i: (i, 0, 0)),
        pl.BlockSpec((_CB, n, d), lambda i: (i, 0, 0)),
        pl.BlockSpec((_CB, m, de), lambda i: (i, 0, 0)),
        pl.BlockSpec(Wv.shape, lambda i: (0, 0)),
        pl.BlockSpec(Ws.shape, lambda i: (0, 0)),
        pl.BlockSpec((8, 128), lambda i: (0, 0)),
    ]
    out_specs = (
        pl.BlockSpec((_CB, n, d), lambda i: (i, 0, 0)),
        pl.BlockSpec((_CB, m, de), lambda i: (i, 0, 0)),
    )
    out_shape = (jax.ShapeDtypeStruct((b, n, d), jnp.float32),
                 jax.ShapeDtypeStruct((b, m, de), jnp.float32))

    flops_per_cell = (2 * 4 * n * n * d + 2 * n * (_DEG * d) * d
                      + 2 * (2 * m) * n * d + 2 * m * (2 * d + de) * de
                      + 12 * (n * d + m * de))
    bytes_accessed = 4 * (a_mean_b.size + s_gather_b.size + v_in.size
                          + e_in.size + Wv.size + Ws.size + 8 * 128
                          + b * n * d + b * m * de)

    return pl.pallas_call(
        body,
        grid=(b // _CB,),
        in_specs=in_specs,
        out_specs=out_specs,
        out_shape=out_shape,
        compiler_params=pltpu.CompilerParams(
            dimension_semantics=("parallel",)),
        cost_estimate=pl.CostEstimate(
            flops=int(b * flops_per_cell),
            transcendentals=int(b * (d + de)),
            bytes_accessed=int(bytes_accessed)),
    )(weight, a_mean_b, s_gather_b, v_in, e_in, Wv, Ws, pp)
